# baseline (speedup 1.0000x reference)
import jax
import jax.numpy as jnp
from jax.experimental import pallas as pl
from jax.experimental.pallas import tpu as pltpu

B, D, C = 2048, 1024, 8192
BC = 512
CH = C // 2
NC = CH // BC

_LN2 = 0.6931471805599453
_INV_LN2 = 1.4426950408889634


def _main_kernel(xt_ref, cen_ref, y_ref, m_out, l_out, t_out, a_out,
                 xs_ref, m_ref, l_ref, t_ref, a_ref):
    p = pl.program_id(0)
    c = pl.program_id(1)

    @pl.when(c == 0)
    def _init():
        xs_ref[...] = xt_ref[...] * (2.0 * _INV_LN2)
        m_ref[...] = jnp.full(m_ref.shape, -jnp.inf, dtype=jnp.float32)
        l_ref[...] = jnp.zeros(l_ref.shape, dtype=jnp.float32)
        t_ref[...] = jnp.zeros(t_ref.shape, dtype=jnp.float32)
        a_ref[...] = jnp.zeros(a_ref.shape, dtype=jnp.float32)

    base = p * CH + c * BC
    cb = cen_ref[...]
    acc = jnp.dot(cb, xs_ref[...], preferred_element_type=jnp.float32)
    c2h = jnp.sum(cb * cb, axis=1, keepdims=True) * _INV_LN2
    h = acc - c2h

    cmax = jnp.max(h, axis=0, keepdims=True)
    row = jax.lax.broadcasted_iota(jnp.int32, (BC, B), 0)
    camax = (jnp.min(jnp.where(h >= cmax, row, C), axis=0,
                     keepdims=True) + base).astype(jnp.float32)
    yloc = y_ref[0] - base
    tsum = jnp.sum(jnp.where(row == yloc, h, 0.0),
                   axis=0, keepdims=True)

    m_old = jnp.max(m_ref[...], axis=0, keepdims=True)
    l_old = jnp.max(l_ref[...], axis=0, keepdims=True)
    a_old = jnp.max(a_ref[...], axis=0, keepdims=True)

    m_new = jnp.maximum(m_old, cmax)
    p_sum = jnp.sum(jnp.exp2(h - m_new), axis=0, keepdims=True)
    l_new = l_old * jnp.exp2(m_old - m_new) + p_sum
    a_new = jnp.where(cmax > m_old, camax, a_old)

    m_ref[...] = jnp.broadcast_to(m_new, m_ref.shape)
    l_ref[...] = jnp.broadcast_to(l_new, l_ref.shape)
    a_ref[...] = jnp.broadcast_to(a_new, a_ref.shape)
    t_ref[...] = t_ref[...] + jnp.broadcast_to(tsum, t_ref.shape)

    @pl.when(c == NC - 1)
    def _fin():
        m_out[...] = jnp.max(m_ref[...], axis=0, keepdims=True).reshape(1, 1, B)
        l_out[...] = jnp.max(l_ref[...], axis=0, keepdims=True).reshape(1, 1, B)
        t_out[...] = jnp.max(t_ref[...], axis=0, keepdims=True).reshape(1, 1, B)
        a_out[...] = jnp.max(a_ref[...], axis=0, keepdims=True).reshape(1, 1, B)


def _merge_kernel(m_ref, l_ref, t_ref, a_ref, y_ref, loss_ref, corr_ref):
    m0, m1 = m_ref[0], m_ref[1]
    l0, l1 = l_ref[0], l_ref[1]
    a0, a1 = a_ref[0], a_ref[1]
    t = t_ref[0] + t_ref[1]
    m = jnp.maximum(m0, m1)
    l = l0 * jnp.exp2(m0 - m) + l1 * jnp.exp2(m1 - m)
    a = jnp.where(m0 >= m1, a0, a1)
    loss_row = (m + jnp.log2(l) - t) * _LN2
    corr_row = (a == y_ref[0].astype(jnp.float32)).astype(jnp.float32)
    loss_ref[...] = jnp.broadcast_to(jnp.sum(loss_row, keepdims=True), (8, 128))
    corr_ref[...] = jnp.broadcast_to(jnp.sum(corr_row, keepdims=True), (8, 128))


@jax.jit
def kernel(x, y, centroids):
    xt = x.T
    y3 = y.astype(jnp.int32).reshape(1, 1, B)
    stat_sds = jax.ShapeDtypeStruct((2, 1, B), jnp.float32)
    m2, l2, t2, a2 = pl.pallas_call(
        _main_kernel,
        grid=(2, NC),
        in_specs=[
            pl.BlockSpec((D, B), lambda p, c: (0, 0)),
            pl.BlockSpec((BC, D), lambda p, c: (p * NC + c, 0)),
            pl.BlockSpec((1, 1, B), lambda p, c: (0, 0, 0)),
        ],
        out_specs=tuple(
            pl.BlockSpec((1, 1, B), lambda p, c: (p, 0, 0)) for _ in range(4)),
        out_shape=(stat_sds,) * 4,
        scratch_shapes=[
            pltpu.VMEM((D, B), jnp.float32),
            pltpu.VMEM((8, B), jnp.float32),
            pltpu.VMEM((8, B), jnp.float32),
            pltpu.VMEM((8, B), jnp.float32),
            pltpu.VMEM((8, B), jnp.float32),
        ],
        compiler_params=pltpu.CompilerParams(
            dimension_semantics=("parallel", "arbitrary"),
            vmem_limit_bytes=100 * 1024 * 1024,
        ),
    )(xt, centroids, y3)

    loss_t, corr_t = pl.pallas_call(
        _merge_kernel,
        out_shape=(jax.ShapeDtypeStruct((8, 128), jnp.float32),
                   jax.ShapeDtypeStruct((8, 128), jnp.float32)),
    )(m2, l2, t2, a2, y3)
    loss = loss_t[0, 0] / B
    score = corr_t[0, 0] / B
    return loss, score


# speedup vs baseline: 1.0025x; 1.0025x over previous
import jax
import jax.numpy as jnp
from jax.experimental import pallas as pl
from jax.experimental.pallas import tpu as pltpu

B, D, C = 2048, 1024, 8192
BC = 512
S = 2
CH = C // 2
NC = CH // (BC * S)

_LN2 = 0.6931471805599453
_INV_LN2 = 1.4426950408889634


def _main_kernel(xt_ref, cen_a_ref, cen_b_ref, y_ref, m_out, l_out, t_out, a_out,
                 xs_ref, m_ref, l_ref, t_ref, a_ref):
    p = pl.program_id(0)
    c = pl.program_id(1)

    @pl.when(c == 0)
    def _init():
        xs_ref[...] = xt_ref[...] * (2.0 * _INV_LN2)
        m_ref[...] = jnp.full(m_ref.shape, -jnp.inf, dtype=jnp.float32)
        l_ref[...] = jnp.zeros(l_ref.shape, dtype=jnp.float32)
        t_ref[...] = jnp.zeros(t_ref.shape, dtype=jnp.float32)
        a_ref[...] = jnp.zeros(a_ref.shape, dtype=jnp.float32)

    def _chunk(cb, base):
        acc = jnp.dot(cb, xs_ref[...], preferred_element_type=jnp.float32)
        c2h = jnp.sum(cb * cb, axis=1, keepdims=True) * _INV_LN2
        h = acc - c2h

        cmax = jnp.max(h, axis=0, keepdims=True)
        row = jax.lax.broadcasted_iota(jnp.int32, (BC, B), 0)
        camax = (jnp.min(jnp.where(h >= cmax, row, C), axis=0,
                         keepdims=True) + base).astype(jnp.float32)
        yloc = y_ref[0] - base
        tsum = jnp.sum(jnp.where(row == yloc, h, 0.0),
                       axis=0, keepdims=True)

        m_old = jnp.max(m_ref[...], axis=0, keepdims=True)
        l_old = jnp.max(l_ref[...], axis=0, keepdims=True)
        a_old = jnp.max(a_ref[...], axis=0, keepdims=True)

        m_new = jnp.maximum(m_old, cmax)
        p_sum = jnp.sum(jnp.exp2(h - m_new), axis=0, keepdims=True)
        l_new = l_old * jnp.exp2(m_old - m_new) + p_sum
        a_new = jnp.where(cmax > m_old, camax, a_old)

        m_ref[...] = jnp.broadcast_to(m_new, m_ref.shape)
        l_ref[...] = jnp.broadcast_to(l_new, l_ref.shape)
        a_ref[...] = jnp.broadcast_to(a_new, a_ref.shape)
        t_ref[...] = t_ref[...] + jnp.broadcast_to(tsum, t_ref.shape)

    base0 = p * CH + c * (S * BC)
    _chunk(cen_a_ref[...], base0)
    _chunk(cen_b_ref[...], base0 + BC)

    @pl.when(c == NC - 1)
    def _fin():
        m_out[...] = jnp.max(m_ref[...], axis=0, keepdims=True).reshape(1, 1, B)
        l_out[...] = jnp.max(l_ref[...], axis=0, keepdims=True).reshape(1, 1, B)
        t_out[...] = jnp.max(t_ref[...], axis=0, keepdims=True).reshape(1, 1, B)
        a_out[...] = jnp.max(a_ref[...], axis=0, keepdims=True).reshape(1, 1, B)


def _merge_kernel(m_ref, l_ref, t_ref, a_ref, y_ref, loss_ref, corr_ref):
    m0, m1 = m_ref[0], m_ref[1]
    l0, l1 = l_ref[0], l_ref[1]
    a0, a1 = a_ref[0], a_ref[1]
    t = t_ref[0] + t_ref[1]
    m = jnp.maximum(m0, m1)
    l = l0 * jnp.exp2(m0 - m) + l1 * jnp.exp2(m1 - m)
    a = jnp.where(m0 >= m1, a0, a1)
    loss_row = (m + jnp.log2(l) - t) * _LN2
    corr_row = (a == y_ref[0].astype(jnp.float32)).astype(jnp.float32)
    loss_ref[...] = jnp.broadcast_to(jnp.sum(loss_row, keepdims=True), (8, 128))
    corr_ref[...] = jnp.broadcast_to(jnp.sum(corr_row, keepdims=True), (8, 128))


@jax.jit
def kernel(x, y, centroids):
    xt = x.T
    y3 = y.astype(jnp.int32).reshape(1, 1, B)
    stat_sds = jax.ShapeDtypeStruct((2, 1, B), jnp.float32)
    m2, l2, t2, a2 = pl.pallas_call(
        _main_kernel,
        grid=(2, NC),
        in_specs=[
            pl.BlockSpec((D, B), lambda p, c: (0, 0)),
            pl.BlockSpec((BC, D), lambda p, c: (p * NC * S + c * S, 0)),
            pl.BlockSpec((BC, D), lambda p, c: (p * NC * S + c * S + 1, 0)),
            pl.BlockSpec((1, 1, B), lambda p, c: (0, 0, 0)),
        ],
        out_specs=tuple(
            pl.BlockSpec((1, 1, B), lambda p, c: (p, 0, 0)) for _ in range(4)),
        out_shape=(stat_sds,) * 4,
        scratch_shapes=[
            pltpu.VMEM((D, B), jnp.float32),
            pltpu.VMEM((8, B), jnp.float32),
            pltpu.VMEM((8, B), jnp.float32),
            pltpu.VMEM((8, B), jnp.float32),
            pltpu.VMEM((8, B), jnp.float32),
        ],
        compiler_params=pltpu.CompilerParams(
            dimension_semantics=("parallel", "arbitrary"),
            vmem_limit_bytes=100 * 1024 * 1024,
        ),
    )(xt, centroids, centroids, y3)

    loss_t, corr_t = pl.pallas_call(
        _merge_kernel,
        out_shape=(jax.ShapeDtypeStruct((8, 128), jnp.float32),
                   jax.ShapeDtypeStruct((8, 128), jnp.float32)),
    )(m2, l2, t2, a2, y3)
    loss = loss_t[0, 0] / B
    score = corr_t[0, 0] / B
    return loss, score


# speedup vs baseline: 1.3855x; 1.3820x over previous
import jax
import jax.numpy as jnp
from jax.experimental import pallas as pl
from jax.experimental.pallas import tpu as pltpu

B, D, C = 2048, 1024, 8192
BC = 512
S = 2
NC = C // (BC * S)

_LN2 = 0.6931471805599453
_INV_LN2 = 1.4426950408889634


def _main_kernel(x_ref, cen_a_ref, cen_b_ref, y_ref, loss_ref, corr_ref,
                 xs_ref, m_ref, l_ref, t_ref, a_ref):
    c = pl.program_id(0)

    @pl.when(c == 0)
    def _init():
        xs_ref[...] = x_ref[...].T * (2.0 * _INV_LN2)
        m_ref[...] = jnp.full(m_ref.shape, -jnp.inf, dtype=jnp.float32)
        l_ref[...] = jnp.zeros(l_ref.shape, dtype=jnp.float32)
        t_ref[...] = jnp.zeros(t_ref.shape, dtype=jnp.float32)
        a_ref[...] = jnp.zeros(a_ref.shape, dtype=jnp.float32)

    def _chunk(cb, base):
        acc = jnp.dot(cb, xs_ref[...], preferred_element_type=jnp.float32)
        c2h = jnp.sum(cb * cb, axis=1, keepdims=True) * _INV_LN2
        h = acc - c2h

        cmax = jnp.max(h, axis=0, keepdims=True)

        m_old = jnp.max(m_ref[...], axis=0, keepdims=True)
        l_old = jnp.max(l_ref[...], axis=0, keepdims=True)
        a_old = jnp.max(a_ref[...], axis=0, keepdims=True)

        m_new = jnp.maximum(m_old, cmax)
        p = jnp.exp2(h - m_new)
        p_sum = jnp.sum(p, axis=0, keepdims=True)
        l_new = l_old * jnp.exp2(m_old - m_new) + p_sum

        row = jax.lax.broadcasted_iota(jnp.int32, (BC, B), 0)
        camax = (jnp.min(jnp.where(p >= 1.0, row, C), axis=0,
                         keepdims=True) + base).astype(jnp.float32)
        a_new = jnp.where(cmax > m_old, camax, a_old)

        yloc = y_ref[0] - base
        tsum = jnp.sum(jnp.where(row == yloc, h, 0.0),
                       axis=0, keepdims=True)

        m_ref[...] = jnp.broadcast_to(m_new, m_ref.shape)
        l_ref[...] = jnp.broadcast_to(l_new, l_ref.shape)
        a_ref[...] = jnp.broadcast_to(a_new, a_ref.shape)
        t_ref[...] = t_ref[...] + jnp.broadcast_to(tsum, t_ref.shape)

    base0 = c * (S * BC)
    _chunk(cen_a_ref[...], base0)
    _chunk(cen_b_ref[...], base0 + BC)

    @pl.when(c == NC - 1)
    def _fin():
        m_c = jnp.max(m_ref[...], axis=0, keepdims=True)
        l_c = jnp.max(l_ref[...], axis=0, keepdims=True)
        t_c = jnp.max(t_ref[...], axis=0, keepdims=True)
        a_c = jnp.max(a_ref[...], axis=0, keepdims=True)
        loss_row = (m_c + jnp.log2(l_c) - t_c) * _LN2
        corr_row = (a_c == y_ref[0].astype(jnp.float32)).astype(jnp.float32)
        ls = jnp.sum(loss_row, keepdims=True)
        cs = jnp.sum(corr_row, keepdims=True)
        loss_ref[...] = jnp.broadcast_to(ls, (8, 128))
        corr_ref[...] = jnp.broadcast_to(cs, (8, 128))


@jax.jit
def kernel(x, y, centroids):
    y3 = y.astype(jnp.int32).reshape(1, 1, B)
    loss_t, corr_t = pl.pallas_call(
        _main_kernel,
        grid=(NC,),
        in_specs=[
            pl.BlockSpec((B, D), lambda c: (0, 0)),
            pl.BlockSpec((BC, D), lambda c: (c * S, 0)),
            pl.BlockSpec((BC, D), lambda c: (c * S + 1, 0)),
            pl.BlockSpec((1, 1, B), lambda c: (0, 0, 0)),
        ],
        out_specs=(pl.BlockSpec((8, 128), lambda c: (0, 0)),
                   pl.BlockSpec((8, 128), lambda c: (0, 0))),
        out_shape=(jax.ShapeDtypeStruct((8, 128), jnp.float32),
                   jax.ShapeDtypeStruct((8, 128), jnp.float32)),
        scratch_shapes=[
            pltpu.VMEM((D, B), jnp.float32),
            pltpu.VMEM((8, B), jnp.float32),
            pltpu.VMEM((8, B), jnp.float32),
            pltpu.VMEM((8, B), jnp.float32),
            pltpu.VMEM((8, B), jnp.float32),
        ],
        compiler_params=pltpu.CompilerParams(
            dimension_semantics=("arbitrary",),
            vmem_limit_bytes=100 * 1024 * 1024,
        ),
    )(x, centroids, centroids, y3)
    loss = loss_t[0, 0] / B
    score = corr_t[0, 0] / B
    return loss, score
